# baseline (speedup 1.0000x reference)
import functools

import jax
import jax.numpy as jnp
from jax import lax
from jax.experimental import pallas as pl
from jax.experimental.pallas import tpu as pltpu
from jax.experimental.pallas import tpu_sc as plsc

NC, NS, L = 2, 16, 16
NW = NC * NS
H = 128
J = H // L
T = 128
PT = 512 * 2
TP = T + 1
EPS = 1e-12


def _rsqrt(x):
    xi = lax.bitcast_convert_type(x, jnp.int32)
    yi = jnp.int32(0x5F3759DF) - lax.shift_right_arithmetic(xi, 1)
    y = lax.bitcast_convert_type(yi, jnp.float32)
    for _ in range(2):
        y = y * (1.5 - 0.5 * x * y * y)
    return y


def _body(n_chunks, wids_hbm, ptids_hbm, word_hbm, pt_hbm, gb_hbm, out_hbm,
          rows, ptrows, widsb, ptidsb, gbv, slab, stats,
          sem_g, sem_p, sem_s, sem_i, sem_t):
    wid = lax.axis_index("s") * NC + lax.axis_index("c")

    pltpu.async_copy(gb_hbm, gbv, sem_t).wait()

    def start_ids(g, b):
        c = wid * n_chunks + g
        pltpu.async_copy(wids_hbm.at[c], widsb.at[b], sem_i[b])
        pltpu.async_copy(ptids_hbm.at[c], ptidsb.at[b], sem_i[b])

    def wait_ids(b):
        pltpu.make_async_copy(wids_hbm.at[0], widsb.at[b], sem_i[b]).wait()
        pltpu.make_async_copy(ptids_hbm.at[0], ptidsb.at[b], sem_i[b]).wait()

    def start_gathers(b):
        pltpu.async_copy(word_hbm.at[widsb.at[b]], rows.at[b], sem_g[b])
        pltpu.async_copy(pt_hbm.at[ptidsb.at[b]], ptrows.at[b], sem_p[b])

    def wait_gathers(b):
        pltpu.make_async_copy(
            word_hbm.at[widsb.at[b]], rows.at[b], sem_g[b]).wait()
        pltpu.make_async_copy(
            pt_hbm.at[ptidsb.at[b]], ptrows.at[b], sem_p[b]).wait()

    def compute_chunk(b):
        rowsb = rows.at[b]
        ptrowsb = ptrows.at[b]
        gs = [gbv[0, pl.ds(L * j, L)] for j in range(J)]
        bs = [gbv[1, pl.ds(L * j, L)] for j in range(J)]
        lanes = lax.iota(jnp.int32, L)
        fj = [(L * j + lanes) * TP for j in range(J)]

        @plsc.parallel_loop(0, T, 1, unroll=4)
        def _(t):
            tb = jnp.full((L,), t, jnp.int32)
            e = [rowsb[t, pl.ds(L * j, L)] + ptrowsb[t, pl.ds(L * j, L)]
                 for j in range(J)]
            for j in range(J):
                rowsb[t, pl.ds(L * j, L)] = e[j]
                plsc.store_scatter(slab, [fj[j] + tb], e[j])

        @plsc.parallel_loop(0, T // L, 1)
        def _(g):
            def dpair(i, carry):
                s0, q0, s1, q1, off = carry
                v0 = slab[pl.ds(off, L)]
                v1 = slab[pl.ds(off + TP, L)]
                return (s0 + v0, q0 + v0 * v0, s1 + v1, q1 + v1 * v1,
                        off + 2 * TP)

            z = jnp.zeros((L,), jnp.float32)
            s0, q0, s1, q1, _o = lax.fori_loop(
                0, H // 2, dpair, (z, z, z, z, g * L), unroll=4)
            mv = (1.0 / H) * (s0 + s1)
            var = (1.0 / H) * (q0 + q1) - mv * mv
            rstd = _rsqrt(var + EPS)
            for k in range(L):
                stats[0, g * L + k, :] = jnp.full((L,), mv[k], jnp.float32)
                stats[1, g * L + k, :] = jnp.full((L,), rstd[k], jnp.float32)

        @plsc.parallel_loop(0, T, 1, unroll=2)
        def _(t):
            bm = stats[0, t, :]
            br = stats[1, t, :]
            e = [rowsb[t, pl.ds(L * j, L)] for j in range(J)]
            for j in range(J):
                rowsb[t, pl.ds(L * j, L)] = (e[j] - bm) * (br * gs[j]) + bs[j]

    def start_scatter(g, b):
        off = (wid * n_chunks + g) * T
        pltpu.async_copy(rows.at[b], out_hbm.at[pl.ds(off, T)], sem_s[b])

    def wait_scatter(b):
        pltpu.make_async_copy(
            rows.at[b], out_hbm.at[pl.ds(0, T)], sem_s[b]).wait()

    start_ids(0, 0)
    start_ids(1, 1)
    wait_ids(0)
    start_gathers(0)

    def loop(g2, _):
        for par in range(2):
            g = g2 * 2 + par
            b = par
            nb = 1 - par
            @pl.when(g + 1 < n_chunks)
            def _():
                @pl.when(g - 1 >= 0)
                def _():
                    wait_scatter(nb)
                wait_ids(nb)
                start_gathers(nb)

            wait_gathers(b)
            compute_chunk(b)
            start_scatter(g, b)

            @pl.when(g + 2 < n_chunks)
            def _():
                start_ids(g + 2, b)
        return 0

    lax.fori_loop(0, n_chunks // 2, loop, 0)
    for b in range(2):
        wait_scatter(b)


def kernel(input_ids, position_ids, token_type_ids, word_emb, pos_emb,
           type_emb, ln_gamma, ln_beta):
    B, Lseq = input_ids.shape
    NT = B * Lseq
    n_chunks = NT // (NW * T)
    C = NW * n_chunks

    wids = input_ids.reshape(C, T).astype(jnp.int32)
    ptids = (position_ids.astype(jnp.int32) * 2
             + token_type_ids.astype(jnp.int32)).reshape(C, T)
    pt_table = (pos_emb[:, None, :] + type_emb[None, :, :]).reshape(PT, H)
    gb = jnp.stack([ln_gamma, ln_beta], axis=0)

    mesh = plsc.VectorSubcoreMesh(
        core_axis_name="c", subcore_axis_name="s",
        num_cores=NC, num_subcores=NS)
    run = pl.kernel(
        functools.partial(_body, n_chunks),
        out_type=jax.ShapeDtypeStruct((NT, H), jnp.float32),
        mesh=mesh,
        compiler_params=pltpu.CompilerParams(needs_layout_passes=False),
        scratch_types=[
            pltpu.VMEM((2, T, H), jnp.float32),
            pltpu.VMEM((2, T, H), jnp.float32),
            pltpu.VMEM((2, T), jnp.int32),
            pltpu.VMEM((2, T), jnp.int32),
            pltpu.VMEM((2, H), jnp.float32),
            pltpu.VMEM((H * TP,), jnp.float32),
            pltpu.VMEM((2, T, L), jnp.float32),
            [pltpu.SemaphoreType.DMA] * 2,
            [pltpu.SemaphoreType.DMA] * 2,
            [pltpu.SemaphoreType.DMA] * 2,
            [pltpu.SemaphoreType.DMA] * 2,
            pltpu.SemaphoreType.DMA,
        ],
    )
    out = run(wids, ptids, word_emb, pt_table, gb)
    return out.reshape(B, Lseq, H)


# speedup vs baseline: 1.2064x; 1.2064x over previous
import functools

import jax
import jax.numpy as jnp
from jax import lax
from jax.experimental import pallas as pl
from jax.experimental.pallas import tpu as pltpu
from jax.experimental.pallas import tpu_sc as plsc

NC, NS, L = 2, 16, 16
NW = NC * NS
H = 128
J = H // L
T = 128
PT = 512 * 2
PP = L + 1
EPS = 1e-12


def _rsqrt(x):
    xi = lax.bitcast_convert_type(x, jnp.int32)
    yi = jnp.int32(0x5F3759DF) - lax.shift_right_arithmetic(xi, 1)
    y = lax.bitcast_convert_type(yi, jnp.float32)
    for _ in range(2):
        y = y * (1.5 - 0.5 * x * y * y)
    return y


def _body(n_chunks, wids_hbm, ptids_hbm, word_hbm, pt_hbm, gb_hbm, out_hbm,
          ptshared, rows, ptrows, widsb, ptidsb, gbv, sbuf, qbuf, stats,
          sem_g, sem_p, sem_s, sem_i, sem_t):
    wid = lax.axis_index("s") * NC + lax.axis_index("c")

    @pl.when(lax.axis_index("s") == 0)
    def _():
        pltpu.async_copy(pt_hbm, ptshared, sem_t).wait()
    pltpu.async_copy(gb_hbm, gbv, sem_t).wait()
    plsc.subcore_barrier()

    def start_ids(g, b):
        c = wid * n_chunks + g
        pltpu.async_copy(wids_hbm.at[c], widsb.at[b], sem_i[b])
        pltpu.async_copy(ptids_hbm.at[c], ptidsb.at[b], sem_i[b])

    def wait_ids(b):
        pltpu.make_async_copy(wids_hbm.at[0], widsb.at[b], sem_i[b]).wait()
        pltpu.make_async_copy(ptids_hbm.at[0], ptidsb.at[b], sem_i[b]).wait()

    def start_gathers(b):
        pltpu.async_copy(word_hbm.at[widsb.at[b]], rows.at[b], sem_g[b])
        pltpu.async_copy(ptshared.at[ptidsb.at[b]], ptrows.at[b], sem_p[b])

    def wait_gathers(b):
        pltpu.make_async_copy(
            word_hbm.at[widsb.at[b]], rows.at[b], sem_g[b]).wait()
        pltpu.make_async_copy(
            ptshared.at[ptidsb.at[b]], ptrows.at[b], sem_p[b]).wait()

    def compute_chunk(b):
        rowsb = rows.at[b]
        ptrowsb = ptrows.at[b]
        gs = [gbv[0, pl.ds(L * j, L)] for j in range(J)]
        bs = [gbv[1, pl.ds(L * j, L)] for j in range(J)]
        lanes = lax.iota(jnp.int32, L)

        @plsc.parallel_loop(0, T, 1, unroll=4)
        def _(t):
            e = [rowsb[t, pl.ds(L * j, L)] + ptrowsb[t, pl.ds(L * j, L)]
                 for j in range(J)]
            for j in range(J):
                rowsb[t, pl.ds(L * j, L)] = e[j]
            sp = ((e[0] + e[1]) + (e[2] + e[3])) + ((e[4] + e[5]) + (e[6] + e[7]))
            q = [ej * ej for ej in e]
            qp = ((q[0] + q[1]) + (q[2] + q[3])) + ((q[4] + q[5]) + (q[6] + q[7]))
            sbuf[pl.ds(t * PP, L)] = sp
            qbuf[pl.ds(t * PP, L)] = qp

        @plsc.parallel_loop(0, T // L, 1)
        def _(g):
            iv0 = (g * L + lanes) * PP

            def dred(i, carry):
                accs, accq, iv = carry
                vs = plsc.load_gather(sbuf, [iv])
                vq = plsc.load_gather(qbuf, [iv])
                return accs + vs, accq + vq, iv + 1

            z = jnp.zeros((L,), jnp.float32)
            accs, accq, _iv = lax.fori_loop(
                0, L, dred, (z, z, iv0), unroll=4)
            mv = (1.0 / H) * accs
            var = (1.0 / H) * accq - mv * mv
            rstd = _rsqrt(var + EPS)
            for k in range(L):
                stats[0, g * L + k, :] = jnp.full((L,), mv[k], jnp.float32)
                stats[1, g * L + k, :] = jnp.full((L,), rstd[k], jnp.float32)

        @plsc.parallel_loop(0, T, 1, unroll=2)
        def _(t):
            bm = stats[0, t, :]
            br = stats[1, t, :]
            e = [rowsb[t, pl.ds(L * j, L)] for j in range(J)]
            for j in range(J):
                rowsb[t, pl.ds(L * j, L)] = (e[j] - bm) * (br * gs[j]) + bs[j]

    def start_scatter(g, b):
        off = (wid * n_chunks + g) * T
        pltpu.async_copy(rows.at[b], out_hbm.at[pl.ds(off, T)], sem_s[b])

    def wait_scatter(b):
        pltpu.make_async_copy(
            rows.at[b], out_hbm.at[pl.ds(0, T)], sem_s[b]).wait()

    start_ids(0, 0)
    start_ids(1, 1)
    wait_ids(0)
    start_gathers(0)

    def loop(g2, _):
        for par in range(2):
            g = g2 * 2 + par
            b = par
            nb = 1 - par
            @pl.when(g + 1 < n_chunks)
            def _():
                @pl.when(g - 1 >= 0)
                def _():
                    wait_scatter(nb)
                wait_ids(nb)
                start_gathers(nb)

            wait_gathers(b)
            compute_chunk(b)
            start_scatter(g, b)

            @pl.when(g + 2 < n_chunks)
            def _():
                start_ids(g + 2, b)
        return 0

    lax.fori_loop(0, n_chunks // 2, loop, 0)
    for b in range(2):
        wait_scatter(b)


def kernel(input_ids, position_ids, token_type_ids, word_emb, pos_emb,
           type_emb, ln_gamma, ln_beta):
    B, Lseq = input_ids.shape
    NT = B * Lseq
    n_chunks = NT // (NW * T)
    C = NW * n_chunks

    wids = input_ids.reshape(C, T).astype(jnp.int32)
    ptids = (position_ids.astype(jnp.int32) * 2
             + token_type_ids.astype(jnp.int32)).reshape(C, T)
    pt_table = (pos_emb[:, None, :] + type_emb[None, :, :]).reshape(PT, H)
    gb = jnp.stack([ln_gamma, ln_beta], axis=0)

    mesh = plsc.VectorSubcoreMesh(
        core_axis_name="c", subcore_axis_name="s",
        num_cores=NC, num_subcores=NS)
    run = pl.kernel(
        functools.partial(_body, n_chunks),
        out_type=jax.ShapeDtypeStruct((NT, H), jnp.float32),
        mesh=mesh,
        compiler_params=pltpu.CompilerParams(needs_layout_passes=False),
        scratch_types=[
            pltpu.VMEM_SHARED((PT, H), jnp.float32),
            pltpu.VMEM((2, T, H), jnp.float32),
            pltpu.VMEM((2, T, H), jnp.float32),
            pltpu.VMEM((2, T), jnp.int32),
            pltpu.VMEM((2, T), jnp.int32),
            pltpu.VMEM((2, H), jnp.float32),
            pltpu.VMEM((T * PP,), jnp.float32),
            pltpu.VMEM((T * PP,), jnp.float32),
            pltpu.VMEM((2, T, L), jnp.float32),
            [pltpu.SemaphoreType.DMA] * 2,
            [pltpu.SemaphoreType.DMA] * 2,
            [pltpu.SemaphoreType.DMA] * 2,
            [pltpu.SemaphoreType.DMA] * 2,
            pltpu.SemaphoreType.DMA,
        ],
    )
    out = run(wids, ptids, word_emb, pt_table, gb)
    return out.reshape(B, Lseq, H)


# speedup vs baseline: 1.2206x; 1.0117x over previous
import functools

import jax
import jax.numpy as jnp
from jax import lax
from jax.experimental import pallas as pl
from jax.experimental.pallas import tpu as pltpu
from jax.experimental.pallas import tpu_sc as plsc

NC, NS, L = 2, 16, 16
NW = NC * NS
H = 128
J = H // L
T = 128
PT = 512 * 2
PP = L + 1
EPS = 1e-12


def _rsqrt(x):
    xi = lax.bitcast_convert_type(x, jnp.int32)
    yi = jnp.int32(0x5F3759DF) - lax.shift_right_arithmetic(xi, 1)
    y = lax.bitcast_convert_type(yi, jnp.float32)
    for _ in range(2):
        y = y * (1.5 - 0.5 * x * y * y)
    return y


def _body(n_chunks, ids_hbm, word_hbm, pt_hbm, gb_hbm, out_hbm,
          ptshared, rows, ptrows, idsb, gbv, sbuf, qbuf, stats,
          sem_g, sem_p, sem_s, sem_i, sem_t):
    wid = lax.axis_index("s") * NC + lax.axis_index("c")

    @pl.when(lax.axis_index("s") == 0)
    def _():
        pltpu.async_copy(pt_hbm, ptshared, sem_t).wait()
    pltpu.async_copy(gb_hbm, gbv, sem_t).wait()
    plsc.subcore_barrier()

    def start_ids(g, b):
        c = wid * n_chunks + g
        pltpu.async_copy(ids_hbm.at[c], idsb.at[b], sem_i[b])

    def wait_ids(b):
        pltpu.make_async_copy(ids_hbm.at[0], idsb.at[b], sem_i[b]).wait()

    def start_gathers(b):
        pltpu.async_copy(word_hbm.at[idsb.at[b, 0]], rows.at[b], sem_g[b])
        pltpu.async_copy(ptshared.at[idsb.at[b, 1]], ptrows.at[b], sem_p[b])

    def wait_gathers(b):
        pltpu.make_async_copy(
            word_hbm.at[idsb.at[b, 0]], rows.at[b], sem_g[b]).wait()
        pltpu.make_async_copy(
            ptshared.at[idsb.at[b, 1]], ptrows.at[b], sem_p[b]).wait()

    def compute_chunk(b):
        rowsb = rows.at[b]
        ptrowsb = ptrows.at[b]
        gs = [gbv[0, pl.ds(L * j, L)] for j in range(J)]
        bs = [gbv[1, pl.ds(L * j, L)] for j in range(J)]
        lanes = lax.iota(jnp.int32, L)

        @plsc.parallel_loop(0, T, 1, unroll=4)
        def _(t):
            e = [rowsb[t, pl.ds(L * j, L)] + ptrowsb[t, pl.ds(L * j, L)]
                 for j in range(J)]
            for j in range(J):
                rowsb[t, pl.ds(L * j, L)] = e[j]
            sp = ((e[0] + e[1]) + (e[2] + e[3])) + ((e[4] + e[5]) + (e[6] + e[7]))
            q = [ej * ej for ej in e]
            qp = ((q[0] + q[1]) + (q[2] + q[3])) + ((q[4] + q[5]) + (q[6] + q[7]))
            sbuf[pl.ds(t * PP, L)] = sp
            qbuf[pl.ds(t * PP, L)] = qp

        @plsc.parallel_loop(0, T // L, 1)
        def _(g):
            iv0 = (g * L + lanes) * PP

            def dred(i, carry):
                accs, accq, iv = carry
                vs = plsc.load_gather(sbuf, [iv])
                vq = plsc.load_gather(qbuf, [iv])
                return accs + vs, accq + vq, iv + 1

            z = jnp.zeros((L,), jnp.float32)
            accs, accq, _iv = lax.fori_loop(
                0, L, dred, (z, z, iv0), unroll=4)
            mv = (1.0 / H) * accs
            var = (1.0 / H) * accq - mv * mv
            rstd = _rsqrt(var + EPS)
            for k in range(L):
                stats[0, g * L + k, :] = jnp.full((L,), mv[k], jnp.float32)
                stats[1, g * L + k, :] = jnp.full((L,), rstd[k], jnp.float32)

        @plsc.parallel_loop(0, T, 1, unroll=4)
        def _(t):
            bm = stats[0, t, :]
            br = stats[1, t, :]
            e = [rowsb[t, pl.ds(L * j, L)] for j in range(J)]
            for j in range(J):
                rowsb[t, pl.ds(L * j, L)] = (e[j] - bm) * (br * gs[j]) + bs[j]

    def start_scatter(g, b):
        off = (wid * n_chunks + g) * T
        pltpu.async_copy(rows.at[b], out_hbm.at[pl.ds(off, T)], sem_s[b])

    def wait_scatter(b):
        pltpu.make_async_copy(
            rows.at[b], out_hbm.at[pl.ds(0, T)], sem_s[b]).wait()

    start_ids(0, 0)
    start_ids(1, 1)
    wait_ids(0)
    start_gathers(0)

    def loop(g2, _):
        for par in range(2):
            g = g2 * 2 + par
            b = par
            nb = 1 - par
            @pl.when(g + 1 < n_chunks)
            def _():
                @pl.when(g - 1 >= 0)
                def _():
                    wait_scatter(nb)
                wait_ids(nb)
                start_gathers(nb)

            wait_gathers(b)
            compute_chunk(b)
            start_scatter(g, b)

            @pl.when(g + 2 < n_chunks)
            def _():
                start_ids(g + 2, b)
        return 0

    lax.fori_loop(0, n_chunks // 2, loop, 0)
    for b in range(2):
        wait_scatter(b)


def kernel(input_ids, position_ids, token_type_ids, word_emb, pos_emb,
           type_emb, ln_gamma, ln_beta):
    B, Lseq = input_ids.shape
    NT = B * Lseq
    n_chunks = NT // (NW * T)
    C = NW * n_chunks

    ids = jnp.stack([
        input_ids.reshape(C, T).astype(jnp.int32),
        (position_ids.astype(jnp.int32) * 2
         + token_type_ids.astype(jnp.int32)).reshape(C, T),
    ], axis=1)
    pt_table = (pos_emb[:, None, :] + type_emb[None, :, :]).reshape(PT, H)
    gb = jnp.stack([ln_gamma, ln_beta], axis=0)

    mesh = plsc.VectorSubcoreMesh(
        core_axis_name="c", subcore_axis_name="s",
        num_cores=NC, num_subcores=NS)
    run = pl.kernel(
        functools.partial(_body, n_chunks),
        out_type=jax.ShapeDtypeStruct((NT, H), jnp.float32),
        mesh=mesh,
        compiler_params=pltpu.CompilerParams(needs_layout_passes=False),
        scratch_types=[
            pltpu.VMEM_SHARED((PT, H), jnp.float32),
            pltpu.VMEM((2, T, H), jnp.float32),
            pltpu.VMEM((2, T, H), jnp.float32),
            pltpu.VMEM((2, 2, T), jnp.int32),
            pltpu.VMEM((2, H), jnp.float32),
            pltpu.VMEM((T * PP,), jnp.float32),
            pltpu.VMEM((T * PP,), jnp.float32),
            pltpu.VMEM((2, T, L), jnp.float32),
            [pltpu.SemaphoreType.DMA] * 2,
            [pltpu.SemaphoreType.DMA] * 2,
            [pltpu.SemaphoreType.DMA] * 2,
            [pltpu.SemaphoreType.DMA] * 2,
            pltpu.SemaphoreType.DMA,
        ],
    )
    out = run(ids, word_emb, pt_table, gb)
    return out.reshape(B, Lseq, H)
